# Initial kernel scaffold; baseline (speedup 1.0000x reference)
import jax
import jax.numpy as jnp
from jax.experimental import pallas as pl


def kernel(x_user, x_item, edge_index_u2i, edge_index_i2u, W_u2i, W_i2u):
    raise NotImplementedError("write your pallas kernel here")


# speedup vs baseline: 5.3612x; 5.3612x over previous
import jax
import jax.numpy as jnp
from jax import lax
from jax.experimental import pallas as pl
from jax.experimental.pallas import tpu as pltpu
from jax.experimental.pallas import tpu_sc as plsc

_N = 10000
_NP = 10240
_D = 128
_E = 320000
_NC = 2
_NS = 16
_C = 80
_EPT = _E // _NS
_NCH = _EPT // _C
_RPT = _NP // _NS


def _mm_body(x_ref, w_ref, y_ref):
    y_ref[...] = lax.dot_general(
        x_ref[...], w_ref[0],
        dimension_numbers=(((1,), (0,)), ((), ())),
        preferred_element_type=jnp.float32,
        precision=lax.Precision.HIGHEST,
    )


def _tc_matmul(x_cat, w_stack):
    nb = 5
    blk = _N // nb
    return pl.pallas_call(
        _mm_body,
        grid=(2, nb),
        in_specs=[
            pl.BlockSpec((blk, _D), lambda r, b: (r * nb + b, 0)),
            pl.BlockSpec((1, _D, _D), lambda r, b: (r, 0, 0)),
        ],
        out_specs=pl.BlockSpec((blk, _D), lambda r, b: (r * nb + b, 0)),
        out_shape=jax.ShapeDtypeStruct((2 * _N, _D), jnp.float32),
    )(x_cat, w_stack)


def _sc_body(y_ref, src_ref, dst_ref, zero_ref, out_ref,
             sidx, didx, rows, agg, sem_i, sem_g):
    c = lax.axis_index("c")
    s = lax.axis_index("s")
    rb = s * _RPT
    pltpu.sync_copy(zero_ref.at[pl.ds(rb, _RPT)], agg.at[pl.ds(rb, _RPT)])
    plsc.subcore_barrier()

    ebase = c * _E + s * _EPT

    def chunk(i, carry):
        eb = ebase + i * _C
        cp_s = pltpu.async_copy(src_ref.at[pl.ds(eb, _C)], sidx, sem_i)
        cp_d = pltpu.async_copy(dst_ref.at[pl.ds(eb, _C)], didx, sem_i)
        cp_s.wait()
        cp_d.wait()
        pltpu.async_copy(y_ref.at[sidx], rows, sem_g).wait()
        pltpu.sync_copy(rows, agg.at[didx], add=True)
        return carry

    lax.fori_loop(0, _NCH, chunk, 0)
    plsc.subcore_barrier()
    pltpu.sync_copy(agg.at[pl.ds(rb, _RPT)],
                    out_ref.at[pl.ds(c * _NP + rb, _RPT)])


_sc_scatter = pl.kernel(
    _sc_body,
    out_type=jax.ShapeDtypeStruct((2 * _NP, _D), jnp.float32),
    mesh=plsc.VectorSubcoreMesh(core_axis_name="c", subcore_axis_name="s",
                                num_cores=_NC, num_subcores=_NS),
    scratch_types=[
        pltpu.VMEM((_C,), jnp.int32),
        pltpu.VMEM((_C,), jnp.int32),
        pltpu.VMEM((_C, _D), jnp.float32),
        pltpu.VMEM_SHARED((_NP, _D), jnp.float32),
        pltpu.SemaphoreType.DMA,
        pltpu.SemaphoreType.DMA,
    ],
)


def kernel(x_user, x_item, edge_index_u2i, edge_index_i2u, W_u2i, W_i2u):
    x_cat = jnp.concatenate([x_user, x_item], axis=0)
    w_stack = jnp.stack([W_u2i, W_i2u], axis=0)
    y_cat = _tc_matmul(x_cat, w_stack)

    src_all = jnp.concatenate([edge_index_u2i[0], edge_index_i2u[0] + _N])
    dst_all = jnp.concatenate([edge_index_u2i[1], edge_index_i2u[1]])
    zeros = jnp.zeros((_NP, _D), jnp.float32)
    out_cat = _sc_scatter(y_cat, src_all, dst_all, zeros)

    out_item = out_cat[:_N]
    out_user = out_cat[_NP:_NP + _N]
    return (out_user, out_item)


# speedup vs baseline: 11.3872x; 2.1240x over previous
import jax
import jax.numpy as jnp
from jax import lax
from jax.experimental import pallas as pl
from jax.experimental.pallas import tpu as pltpu
from jax.experimental.pallas import tpu_sc as plsc

_N = 10000
_NP = 10240
_D = 128
_E = 320000
_NC = 2
_NS = 16
_C = 128
_EPT = _E // _NS
_NCH = _EPT // _C
_CT = _EPT - _NCH * _C
_RPT = _NP // _NS


def _mm_body(x_ref, w_ref, y_ref):
    y_ref[...] = lax.dot_general(
        x_ref[...], w_ref[0],
        dimension_numbers=(((1,), (0,)), ((), ())),
        preferred_element_type=jnp.float32,
        precision=lax.Precision.HIGHEST,
    )


def _tc_matmul(x_cat, w_stack):
    nb = 5
    blk = _N // nb
    return pl.pallas_call(
        _mm_body,
        grid=(2, nb),
        in_specs=[
            pl.BlockSpec((blk, _D), lambda r, b: (r * nb + b, 0)),
            pl.BlockSpec((1, _D, _D), lambda r, b: (r, 0, 0)),
        ],
        out_specs=pl.BlockSpec((blk, _D), lambda r, b: (r * nb + b, 0)),
        out_shape=jax.ShapeDtypeStruct((2 * _N, _D), jnp.float32),
    )(x_cat, w_stack)


def _sc_body(y_ref, src_ref, dst_ref, zero_ref, out_ref,
             sidx, didx, rows, tsidx, tdidx, trows, agg,
             sem_i, sem_g, sem_t):
    c = lax.axis_index("c")
    s = lax.axis_index("s")
    rb = s * _RPT
    ebase = c * _E + s * _EPT

    def load_idx(k, b):
        eb = ebase + k * _C
        pltpu.async_copy(src_ref.at[pl.ds(eb, _C)], sidx[b], sem_i[b])
        pltpu.async_copy(dst_ref.at[pl.ds(eb, _C)], didx[b], sem_i[b])

    def wait_idx(b):
        pltpu.make_async_copy(src_ref.at[pl.ds(0, _C)], sidx[b], sem_i[b]).wait()
        pltpu.make_async_copy(dst_ref.at[pl.ds(0, _C)], didx[b], sem_i[b]).wait()

    def start_gather(b, g):
        pltpu.async_copy(y_ref.at[sidx[b]], rows[g], sem_g[g])

    def wait_gather(g):
        pltpu.make_async_copy(y_ref.at[sidx[0]], rows[g], sem_g[g]).wait()

    for b in range(4):
        load_idx(b, b)
    pltpu.sync_copy(zero_ref.at[pl.ds(rb, _RPT)], agg.at[pl.ds(rb, _RPT)])
    plsc.subcore_barrier()
    wait_idx(0)
    start_gather(0, 0)
    wait_idx(1)
    start_gather(1, 1)

    def quad(j, carry):
        k0 = 4 * j
        for u in range(4):
            k = k0 + u
            g = u % 2
            wait_gather(g)
            pltpu.sync_copy(rows[g], agg.at[didx[u]], add=True)

            @pl.when(k + 4 < _NCH)
            def _():
                load_idx(k + 4, u)

            if u < 2:
                wait_idx(u + 2)
                start_gather(u + 2, g)
            else:

                @pl.when(k + 2 < _NCH)
                def _():
                    wait_idx(u - 2)
                    start_gather(u - 2, g)

        return carry

    lax.fori_loop(0, _NCH // 4, quad, 0)

    et = ebase + _NCH * _C
    pltpu.async_copy(src_ref.at[pl.ds(et, _CT)], tsidx, sem_t)
    pltpu.async_copy(dst_ref.at[pl.ds(et, _CT)], tdidx, sem_t)
    pltpu.make_async_copy(src_ref.at[pl.ds(0, _CT)], tsidx, sem_t).wait()
    pltpu.make_async_copy(dst_ref.at[pl.ds(0, _CT)], tdidx, sem_t).wait()
    pltpu.async_copy(y_ref.at[tsidx], trows, sem_t).wait()
    pltpu.sync_copy(trows, agg.at[tdidx], add=True)

    plsc.subcore_barrier()
    pltpu.sync_copy(agg.at[pl.ds(rb, _RPT)],
                    out_ref.at[pl.ds(c * _NP + rb, _RPT)])


_sc_scatter = pl.kernel(
    _sc_body,
    out_type=jax.ShapeDtypeStruct((2 * _NP, _D), jnp.float32),
    mesh=plsc.VectorSubcoreMesh(core_axis_name="c", subcore_axis_name="s",
                                num_cores=_NC, num_subcores=_NS),
    scratch_types=[
        [pltpu.VMEM((_C,), jnp.int32) for _ in range(4)],
        [pltpu.VMEM((_C,), jnp.int32) for _ in range(4)],
        [pltpu.VMEM((_C, _D), jnp.float32) for _ in range(2)],
        pltpu.VMEM((_CT,), jnp.int32),
        pltpu.VMEM((_CT,), jnp.int32),
        pltpu.VMEM((_CT, _D), jnp.float32),
        pltpu.VMEM_SHARED((_NP, _D), jnp.float32),
        [pltpu.SemaphoreType.DMA for _ in range(4)],
        [pltpu.SemaphoreType.DMA for _ in range(2)],
        pltpu.SemaphoreType.DMA,
    ],
)


def kernel(x_user, x_item, edge_index_u2i, edge_index_i2u, W_u2i, W_i2u):
    x_cat = jnp.concatenate([x_user, x_item], axis=0)
    w_stack = jnp.stack([W_u2i, W_i2u], axis=0)
    y_cat = _tc_matmul(x_cat, w_stack)

    src_all = jnp.concatenate([edge_index_u2i[0], edge_index_i2u[0] + _N])
    dst_all = jnp.concatenate([edge_index_u2i[1], edge_index_i2u[1]])
    zeros = jnp.zeros((_NP, _D), jnp.float32)
    out_cat = _sc_scatter(y_cat, src_all, dst_all, zeros)

    out_item = out_cat[:_N]
    out_user = out_cat[_NP:_NP + _N]
    return (out_user, out_item)


# speedup vs baseline: 12.8245x; 1.1262x over previous
import jax
import jax.numpy as jnp
from jax import lax
from jax.experimental import pallas as pl
from jax.experimental.pallas import tpu as pltpu
from jax.experimental.pallas import tpu_sc as plsc

_N = 10000
_D = 128
_E = 320000
_NC = 2
_NS = 16
_C = 128
_EPT = _E // _NS
_NCH = _EPT // _C
_CT = _EPT - _NCH * _C
_RPT = 640


def _mm_body(xu_ref, wu_ref, xi_ref, wi_ref, yu_ref, yi_ref):
    yu_ref[...] = lax.dot_general(
        xu_ref[...], wu_ref[...], dimension_numbers=(((1,), (0,)), ((), ())),
        preferred_element_type=jnp.float32, precision=lax.Precision.HIGHEST)
    yi_ref[...] = lax.dot_general(
        xi_ref[...], wi_ref[...], dimension_numbers=(((1,), (0,)), ((), ())),
        preferred_element_type=jnp.float32, precision=lax.Precision.HIGHEST)


def _tc_matmul(x_user, W_u2i, x_item, W_i2u):
    nb = 5
    blk = _N // nb
    return pl.pallas_call(
        _mm_body,
        grid=(nb,),
        in_specs=[
            pl.BlockSpec((blk, _D), lambda b: (b, 0)),
            pl.BlockSpec((_D, _D), lambda b: (0, 0)),
            pl.BlockSpec((blk, _D), lambda b: (b, 0)),
            pl.BlockSpec((_D, _D), lambda b: (0, 0)),
        ],
        out_specs=[
            pl.BlockSpec((blk, _D), lambda b: (b, 0)),
            pl.BlockSpec((blk, _D), lambda b: (b, 0)),
        ],
        out_shape=[
            jax.ShapeDtypeStruct((_N, _D), jnp.float32),
            jax.ShapeDtypeStruct((_N, _D), jnp.float32),
        ],
    )(x_user, W_u2i, x_item, W_i2u)


def _sc_body(y_u, y_i, s_u2i, d_u2i, s_i2u, d_i2u, zero_ref,
             out_item, out_user,
             sidx, didx, rows, tsidx, tdidx, trows, agg,
             sem_i, sem_g, sem_t):
    c = lax.axis_index("c")
    s = lax.axis_index("s")
    rb = s * _RPT
    ebase = s * _EPT

    def run_rel(y_ref, src_ref, dst_ref, out_ref):
        def load_idx(k, b):
            eb = ebase + k * _C
            pltpu.async_copy(src_ref.at[pl.ds(eb, _C)], sidx[b], sem_i[b])
            pltpu.async_copy(dst_ref.at[pl.ds(eb, _C)], didx[b], sem_i[b])

        def wait_idx(b):
            pltpu.make_async_copy(
                src_ref.at[pl.ds(0, _C)], sidx[b], sem_i[b]).wait()
            pltpu.make_async_copy(
                dst_ref.at[pl.ds(0, _C)], didx[b], sem_i[b]).wait()

        def start_gather(b, g):
            pltpu.async_copy(y_ref.at[sidx[b]], rows[g], sem_g[g])

        def wait_gather(g):
            pltpu.make_async_copy(y_ref.at[sidx[0]], rows[g], sem_g[g]).wait()

        for b in range(4):
            load_idx(b, b)
        @pl.when(s < _NS - 1)
        def _():
            pltpu.sync_copy(zero_ref.at[pl.ds(rb, _RPT)],
                            agg.at[pl.ds(rb, _RPT)])

        @pl.when(s == _NS - 1)
        def _():
            pltpu.sync_copy(zero_ref.at[pl.ds((_NS - 1) * _RPT, _N - (_NS - 1) * _RPT)],
                            agg.at[pl.ds((_NS - 1) * _RPT, _N - (_NS - 1) * _RPT)])

        plsc.subcore_barrier()
        wait_idx(0)
        start_gather(0, 0)
        wait_idx(1)
        start_gather(1, 1)

        def quad(j, carry):
            k0 = 4 * j
            for u in range(4):
                k = k0 + u
                g = u % 2
                wait_gather(g)
                pltpu.sync_copy(rows[g], agg.at[didx[u]], add=True)

                @pl.when(k + 4 < _NCH)
                def _():
                    load_idx(k + 4, u)

                if u < 2:
                    wait_idx(u + 2)
                    start_gather(u + 2, g)
                else:

                    @pl.when(k + 2 < _NCH)
                    def _():
                        wait_idx(u - 2)
                        start_gather(u - 2, g)

            return carry

        lax.fori_loop(0, _NCH // 4, quad, 0)

        et = ebase + _NCH * _C
        pltpu.async_copy(src_ref.at[pl.ds(et, _CT)], tsidx, sem_t)
        pltpu.async_copy(dst_ref.at[pl.ds(et, _CT)], tdidx, sem_t)
        pltpu.make_async_copy(src_ref.at[pl.ds(0, _CT)], tsidx, sem_t).wait()
        pltpu.make_async_copy(dst_ref.at[pl.ds(0, _CT)], tdidx, sem_t).wait()
        pltpu.async_copy(y_ref.at[tsidx], trows, sem_t).wait()
        pltpu.sync_copy(trows, agg.at[tdidx], add=True)

        plsc.subcore_barrier()
        @pl.when(s < _NS - 1)
        def _():
            pltpu.sync_copy(agg.at[pl.ds(rb, _RPT)],
                            out_ref.at[pl.ds(rb, _RPT)])

        @pl.when(s == _NS - 1)
        def _():
            pltpu.sync_copy(agg.at[pl.ds((_NS - 1) * _RPT, _N - (_NS - 1) * _RPT)],
                            out_ref.at[pl.ds((_NS - 1) * _RPT, _N - (_NS - 1) * _RPT)])

    @pl.when(c == 0)
    def _():
        run_rel(y_u, s_u2i, d_u2i, out_item)

    @pl.when(c == 1)
    def _():
        run_rel(y_i, s_i2u, d_i2u, out_user)


_sc_scatter = pl.kernel(
    _sc_body,
    out_type=(
        jax.ShapeDtypeStruct((_N, _D), jnp.float32),
        jax.ShapeDtypeStruct((_N, _D), jnp.float32),
    ),
    mesh=plsc.VectorSubcoreMesh(core_axis_name="c", subcore_axis_name="s",
                                num_cores=_NC, num_subcores=_NS),
    scratch_types=[
        [pltpu.VMEM((_C,), jnp.int32) for _ in range(4)],
        [pltpu.VMEM((_C,), jnp.int32) for _ in range(4)],
        [pltpu.VMEM((_C, _D), jnp.float32) for _ in range(2)],
        pltpu.VMEM((_CT,), jnp.int32),
        pltpu.VMEM((_CT,), jnp.int32),
        pltpu.VMEM((_CT, _D), jnp.float32),
        pltpu.VMEM_SHARED((_N, _D), jnp.float32),
        [pltpu.SemaphoreType.DMA for _ in range(4)],
        [pltpu.SemaphoreType.DMA for _ in range(2)],
        pltpu.SemaphoreType.DMA,
    ],
)


def kernel(x_user, x_item, edge_index_u2i, edge_index_i2u, W_u2i, W_i2u):
    y_user, y_item = _tc_matmul(x_user, W_u2i, x_item, W_i2u)
    zeros = jnp.zeros((_N, _D), jnp.float32)
    out_item, out_user = _sc_scatter(
        y_user, y_item,
        edge_index_u2i[0], edge_index_u2i[1],
        edge_index_i2u[0], edge_index_i2u[1],
        zeros)
    return (out_user, out_item)


# speedup vs baseline: 13.4914x; 1.0520x over previous
import jax
import jax.numpy as jnp
from jax import lax
from jax.experimental import pallas as pl
from jax.experimental.pallas import tpu as pltpu
from jax.experimental.pallas import tpu_sc as plsc

_N = 10000
_D = 128
_E = 320000
_NC = 2
_NS = 16
_C = 128
_EPT = _E // _NS
_NCH = _EPT // _C
_CT = _EPT - _NCH * _C
_RPT = 640


def _mm_body(xu_ref, wu_ref, xi_ref, wi_ref, yu_ref, yi_ref):
    yu_ref[...] = lax.dot_general(
        xu_ref[...], wu_ref[...], dimension_numbers=(((1,), (0,)), ((), ())),
        preferred_element_type=jnp.float32, precision=lax.Precision.DEFAULT)
    yi_ref[...] = lax.dot_general(
        xi_ref[...], wi_ref[...], dimension_numbers=(((1,), (0,)), ((), ())),
        preferred_element_type=jnp.float32, precision=lax.Precision.DEFAULT)


def _tc_matmul(x_user, W_u2i, x_item, W_i2u):
    nb = 5
    blk = _N // nb
    return pl.pallas_call(
        _mm_body,
        grid=(nb,),
        in_specs=[
            pl.BlockSpec((blk, _D), lambda b: (b, 0)),
            pl.BlockSpec((_D, _D), lambda b: (0, 0)),
            pl.BlockSpec((blk, _D), lambda b: (b, 0)),
            pl.BlockSpec((_D, _D), lambda b: (0, 0)),
        ],
        out_specs=[
            pl.BlockSpec((blk, _D), lambda b: (b, 0)),
            pl.BlockSpec((blk, _D), lambda b: (b, 0)),
        ],
        out_shape=[
            jax.ShapeDtypeStruct((_N, _D), jnp.float32),
            jax.ShapeDtypeStruct((_N, _D), jnp.float32),
        ],
    )(x_user, W_u2i, x_item, W_i2u)


def _sc_body(y_u, y_i, e_u2i, e_i2u, zero_ref,
             out_item, out_user,
             sidx, didx, rows, tsidx, tdidx, trows, agg,
             sem_i, sem_g, sem_t):
    c = lax.axis_index("c")
    s = lax.axis_index("s")
    rb = s * _RPT
    ebase = s * _EPT

    def run_rel(y_ref, edge_ref, out_ref):
        def load_idx(k, b):
            eb = ebase + k * _C
            pltpu.async_copy(edge_ref.at[pl.ds(eb, _C)], sidx[b], sem_i[b])
            pltpu.async_copy(edge_ref.at[pl.ds(_E + eb, _C)], didx[b],
                             sem_i[b])

        def wait_idx(b):
            pltpu.make_async_copy(
                edge_ref.at[pl.ds(0, _C)], sidx[b], sem_i[b]).wait()
            pltpu.make_async_copy(
                edge_ref.at[pl.ds(0, _C)], didx[b], sem_i[b]).wait()

        def start_gather(b, g):
            pltpu.async_copy(y_ref.at[sidx[b]], rows[g], sem_g[g])

        def wait_gather(g):
            pltpu.make_async_copy(y_ref.at[sidx[0]], rows[g], sem_g[g]).wait()

        for b in range(4):
            load_idx(b, b)
        @pl.when(s < _NS - 1)
        def _():
            pltpu.sync_copy(zero_ref.at[pl.ds(rb, _RPT)],
                            agg.at[pl.ds(rb, _RPT)])

        @pl.when(s == _NS - 1)
        def _():
            pltpu.sync_copy(zero_ref.at[pl.ds((_NS - 1) * _RPT, _N - (_NS - 1) * _RPT)],
                            agg.at[pl.ds((_NS - 1) * _RPT, _N - (_NS - 1) * _RPT)])

        plsc.subcore_barrier()
        wait_idx(0)
        start_gather(0, 0)
        wait_idx(1)
        start_gather(1, 1)

        def quad(j, carry):
            k0 = 4 * j
            for u in range(4):
                k = k0 + u
                g = u % 2
                wait_gather(g)
                pltpu.sync_copy(rows[g], agg.at[didx[u]], add=True)

                @pl.when(k + 4 < _NCH)
                def _():
                    load_idx(k + 4, u)

                if u < 2:
                    wait_idx(u + 2)
                    start_gather(u + 2, g)
                else:

                    @pl.when(k + 2 < _NCH)
                    def _():
                        wait_idx(u - 2)
                        start_gather(u - 2, g)

            return carry

        lax.fori_loop(0, _NCH // 4, quad, 0)

        et = ebase + _NCH * _C
        pltpu.async_copy(edge_ref.at[pl.ds(et, _CT)], tsidx, sem_t)
        pltpu.async_copy(edge_ref.at[pl.ds(_E + et, _CT)], tdidx, sem_t)
        pltpu.make_async_copy(edge_ref.at[pl.ds(0, _CT)], tsidx, sem_t).wait()
        pltpu.make_async_copy(edge_ref.at[pl.ds(0, _CT)], tdidx, sem_t).wait()
        pltpu.async_copy(y_ref.at[tsidx], trows, sem_t).wait()
        pltpu.sync_copy(trows, agg.at[tdidx], add=True)

        plsc.subcore_barrier()
        @pl.when(s < _NS - 1)
        def _():
            pltpu.sync_copy(agg.at[pl.ds(rb, _RPT)],
                            out_ref.at[pl.ds(rb, _RPT)])

        @pl.when(s == _NS - 1)
        def _():
            pltpu.sync_copy(agg.at[pl.ds((_NS - 1) * _RPT, _N - (_NS - 1) * _RPT)],
                            out_ref.at[pl.ds((_NS - 1) * _RPT, _N - (_NS - 1) * _RPT)])

    @pl.when(c == 0)
    def _():
        run_rel(y_u, e_u2i, out_item)

    @pl.when(c == 1)
    def _():
        run_rel(y_i, e_i2u, out_user)


_sc_scatter = pl.kernel(
    _sc_body,
    out_type=(
        jax.ShapeDtypeStruct((_N, _D), jnp.float32),
        jax.ShapeDtypeStruct((_N, _D), jnp.float32),
    ),
    mesh=plsc.VectorSubcoreMesh(core_axis_name="c", subcore_axis_name="s",
                                num_cores=_NC, num_subcores=_NS),
    scratch_types=[
        [pltpu.VMEM((_C,), jnp.int32) for _ in range(4)],
        [pltpu.VMEM((_C,), jnp.int32) for _ in range(4)],
        [pltpu.VMEM((_C, _D), jnp.float32) for _ in range(2)],
        pltpu.VMEM((_CT,), jnp.int32),
        pltpu.VMEM((_CT,), jnp.int32),
        pltpu.VMEM((_CT, _D), jnp.float32),
        pltpu.VMEM_SHARED((_N, _D), jnp.float32),
        [pltpu.SemaphoreType.DMA for _ in range(4)],
        [pltpu.SemaphoreType.DMA for _ in range(2)],
        pltpu.SemaphoreType.DMA,
    ],
)


def kernel(x_user, x_item, edge_index_u2i, edge_index_i2u, W_u2i, W_i2u):
    y_user, y_item = _tc_matmul(x_user, W_u2i, x_item, W_i2u)
    zeros = jnp.zeros((_N, _D), jnp.float32)
    out_item, out_user = _sc_scatter(
        y_user, y_item,
        edge_index_u2i.reshape(2 * _E), edge_index_i2u.reshape(2 * _E),
        zeros)
    return (out_user, out_item)
